# Initial kernel scaffold; baseline (speedup 1.0000x reference)
import jax
import jax.numpy as jnp
from jax.experimental import pallas as pl


def kernel(feature, edge_index, edge_w, layer_weights):
    raise NotImplementedError("write your pallas kernel here")


# speedup vs baseline: 2.7327x; 2.7327x over previous
import functools

import jax
import jax.numpy as jnp
from jax import lax
from jax.experimental import pallas as pl
from jax.experimental.pallas import tpu as pltpu
from jax.experimental.pallas import tpu_sc as plsc

N = 10000
E = 320000
D = 128
N_LAYERS = 2
DROPOUT_PROB = 0.3

NC = 2
NS = 16
NW = NC * NS

NP = 10240
CH = 128
NCH = 80
EPT = NCH * CH
EP = NW * EPT
NROWS = NW * NCH
DEG_ROWS = NROWS // NS

_mesh = plsc.VectorSubcoreMesh(core_axis_name="c", subcore_axis_name="s")
_params = pltpu.CompilerParams(needs_layout_passes=False)


def _rsqrt16(d):
    xi = lax.bitcast_convert_type(d, jnp.int32)
    yi = jnp.int32(0x5F3759DF) - lax.shift_right_logical(xi, 1)
    y = lax.bitcast_convert_type(yi, jnp.float32)
    for _ in range(3):
        y = y * (jnp.float32(1.5) - jnp.float32(0.5) * d * y * y)
    return y


@functools.partial(
    pl.kernel,
    out_type=jax.ShapeDtypeStruct((NROWS, CH), jnp.float32),
    mesh=_mesh,
    compiler_params=_params,
    scratch_types=[
        pltpu.VMEM((DEG_ROWS, CH), jnp.int32),
        pltpu.VMEM((DEG_ROWS, CH), jnp.int32),
        pltpu.VMEM((CH,), jnp.float32),
        pltpu.VMEM((NP,), jnp.float32),
        pltpu.VMEM((NP,), jnp.float32),
        pltpu.VMEM((NCH, CH), jnp.int32),
        pltpu.VMEM((NCH, CH), jnp.int32),
        pltpu.VMEM((NCH, CH), jnp.float32),
        pltpu.VMEM((NCH, CH), jnp.float32),
        pltpu.VMEM((NP // NS,), jnp.float32),
        pltpu.VMEM_SHARED((NP,), jnp.float32),
        pltpu.VMEM_SHARED((NP,), jnp.float32),
    ],
)
def _coeff_kernel(src2d, dst2d, w2, c_out,
                  std, dtd, ones, dot, dit, stf, dtf, wf, cb, zb,
                  deg_o, deg_i):
    cid = lax.axis_index("c")
    sid = lax.axis_index("s")
    wid = cid * NS + sid
    zslice = NP // NS

    def _init(i, _):
        zb[pl.ds(i * 16, 16)] = jnp.zeros((16,), jnp.float32)
        return 0
    lax.fori_loop(0, zslice // 16, _init, 0)

    def _init1(i, _):
        ones[pl.ds(i * 16, 16)] = jnp.full((16,), 1.0, jnp.float32)
        return 0
    lax.fori_loop(0, CH // 16, _init1, 0)

    pltpu.sync_copy(zb, deg_o.at[pl.ds(sid * zslice, zslice)])
    pltpu.sync_copy(zb, deg_i.at[pl.ds(sid * zslice, zslice)])
    plsc.subcore_barrier()

    base = sid * DEG_ROWS
    pltpu.sync_copy(src2d.at[pl.ds(base, DEG_ROWS)], std)
    pltpu.sync_copy(dst2d.at[pl.ds(base, DEG_ROWS)], dtd)

    def _deg(j, _):
        pltpu.sync_copy(ones, deg_o.at[std.at[j]], add=True)
        pltpu.sync_copy(ones, deg_i.at[dtd.at[j]], add=True)
        return 0
    lax.fori_loop(0, DEG_ROWS, _deg, 0)
    plsc.subcore_barrier()

    pltpu.sync_copy(deg_o, dot)
    pltpu.sync_copy(deg_i, dit)

    def _norm(i, _):
        sl = pl.ds(i * 16, 16)
        dot[sl] = _rsqrt16(jnp.maximum(dot[sl], jnp.float32(1.0)))
        dit[sl] = _rsqrt16(jnp.maximum(dit[sl], jnp.float32(1.0)))
        return 0
    lax.fori_loop(0, NP // 16, _norm, 0)

    rbase = wid * NCH
    pltpu.sync_copy(src2d.at[pl.ds(rbase, NCH)], stf)
    pltpu.sync_copy(dst2d.at[pl.ds(rbase, NCH)], dtf)
    pltpu.sync_copy(w2.at[pl.ds(rbase, NCH)], wf)

    def _coef(r, _):
        for g in range(CH // 16):
            sl = pl.ds(g * 16, 16)
            no = plsc.load_gather(dot, [stf[r, sl]])
            ni = plsc.load_gather(dit, [dtf[r, sl]])
            cb[r, sl] = wf[r, sl] * no * ni
        return 0
    lax.fori_loop(0, NCH, _coef, 0)
    pltpu.sync_copy(cb, c_out.at[pl.ds(rbase, NCH)])


@functools.partial(
    pl.kernel,
    out_type=jax.ShapeDtypeStruct((NC * NP, D), jnp.float32),
    mesh=_mesh,
    compiler_params=_params,
    scratch_types=[
        pltpu.VMEM((NCH, CH), jnp.int32),
        pltpu.VMEM((NCH, CH), jnp.int32),
        pltpu.VMEM((NCH, CH), jnp.float32),
        pltpu.VMEM((CH, D), jnp.float32),
        pltpu.VMEM((16, D), jnp.float32),
        pltpu.VMEM_SHARED((NP, D), jnp.float32),
        pltpu.SemaphoreType.DMA,
    ],
)
def _layer_kernel(h, src2d, dst2d, c2, part,
                  st, dt, cf, rows, zrows, agg, gsem):
    cid = lax.axis_index("c")
    sid = lax.axis_index("s")
    wid = cid * NS + sid
    rslice = NP // NS

    base = wid * NCH
    pltpu.sync_copy(src2d.at[pl.ds(base, NCH)], st)
    pltpu.sync_copy(dst2d.at[pl.ds(base, NCH)], dt)
    pltpu.sync_copy(c2.at[pl.ds(base, NCH)], cf)

    @pl.when(cid == 0)
    def _():
        pltpu.sync_copy(h.at[pl.ds(sid * rslice, rslice), :],
                        agg.at[pl.ds(sid * rslice, rslice), :])

    @pl.when(cid != 0)
    def _():
        for r in range(16):
            for v in range(D // 16):
                zrows[r, pl.ds(v * 16, 16)] = jnp.zeros((16,), jnp.float32)

        def _z(i, _):
            pltpu.sync_copy(
                zrows, agg.at[pl.ds(sid * rslice + i * 16, 16), :])
            return 0
        lax.fori_loop(0, rslice // 16, _z, 0)

    plsc.subcore_barrier()

    def _chunk(j, _):
        pltpu.async_copy(h.at[st.at[j]], rows, gsem).wait()

        def _scale(g, _):
            cv = cf[j, pl.ds(g * 16, 16)]
            for k in range(16):
                cbk = cv.at[jnp.full((16,), k, jnp.int32)].get(
                    mode="promise_in_bounds")
                e = g * 16 + k
                for v in range(D // 16):
                    sl = pl.ds(v * 16, 16)
                    rows[e, sl] = rows[e, sl] * cbk
            return 0
        lax.fori_loop(0, CH // 16, _scale, 0)

        pltpu.sync_copy(rows, agg.at[dt.at[j]], add=True)
        return 0
    lax.fori_loop(0, NCH, _chunk, 0)
    plsc.subcore_barrier()

    off = cid * NP + sid * rslice
    pltpu.sync_copy(agg.at[pl.ds(sid * rslice, rslice), :],
                    part.at[pl.ds(off, rslice), :])


def kernel(feature, edge_index, edge_w, layer_weights):
    src = edge_index[0]
    dst = edge_index[1]

    kd = jax.random.key(42)
    drop_size = int((1.0 - DROPOUT_PROB) * E)
    ridx = jax.random.randint(kd, (drop_size,), 0, E, dtype=jnp.int32)
    mask = jnp.zeros((E,), dtype=bool).at[ridx].set(True)
    w = jnp.where(mask, jnp.float32(0.0), edge_w)

    pad = EP - E
    pad_idx = (N + (jnp.arange(pad, dtype=jnp.int32) % (NP - N))).astype(jnp.int32)
    src_p = jnp.concatenate([src, pad_idx])
    dst_p = jnp.concatenate([dst, pad_idx])
    w_p = jnp.concatenate([w, jnp.zeros((pad,), jnp.float32)])

    src2d = src_p.reshape(NROWS, CH)
    dst2d = dst_p.reshape(NROWS, CH)
    w2 = w_p.reshape(NROWS, CH)

    h0 = jnp.pad(feature, ((0, NP - N), (0, 0)))

    c2 = _coeff_kernel(src2d, dst2d, w2)

    embs = [h0]
    h = h0
    for _ in range(N_LAYERS):
        part = _layer_kernel(h, src2d, dst2d, c2)
        h = part[:NP] + part[NP:]
        embs.append(h)

    lw = layer_weights
    out = lw[0] * embs[0] + lw[1] * embs[1] + lw[2] * embs[2]
    return out[:N]
